# baseline (speedup 1.0000x reference)
import functools

import jax
import jax.numpy as jnp
from jax.experimental import pallas as pl
from jax.experimental.pallas import tpu as pltpu

_NS = [1024, 2048, 1536, 512]
_H = 256
_HC = 3 * _H


def _starts(steps):
    s, acc = [], 0
    for v in steps:
        s.append(acc)
        acc += v
    return s, acc


def _win_row(start, last):
    return lambda t: (jnp.clip(t - start, 0, last), 0)


def _win_col(start, last):
    return lambda t: (0, jnp.clip(t - start, 0, last))


def _pwin_row(start, last, period):
    return lambda t: (jnp.clip(t % period - start, 0, last), 0)


def _pwin_col(start, last, period):
    return lambda t: (0, jnp.clip(t % period - start, 0, last))


def _const2(i, k):
    return lambda t, _i=i, _k=k: (_i, _k)


def _rsel3(period, hi):
    return lambda t: (jnp.clip(t // period, 0, hi), 0, 0)


def _attn_math(a, h, hb, nz, ylow, yup, blow, bup, bound_softmax=True):
    acc = None
    if blow is not None:
        acc = jax.lax.dot_general(
            blow, ylow,
            dimension_numbers=(((0,), (0,)), ((), ())),
            preferred_element_type=jnp.float32,
        )
    if bup is not None:
        up = jnp.dot(bup, yup, preferred_element_type=jnp.float32)
        acc = up if acc is None else acc + up
    s_dst = jnp.sum(h * a[1:2, :], axis=1)[None, :]
    s_src = jnp.sum(hb * a[0:1, :], axis=1, keepdims=True)
    e = s_src + s_dst
    e = jnp.maximum(e, 0.2 * e)
    if bound_softmax:
        b = s_src + jnp.max(s_dst)
        b = jnp.maximum(b, 0.2 * b)
        p = jnp.where(nz, jnp.exp(e - b), 0.0)
    else:
        e = jnp.where(nz, e, -1e9)
        m = jnp.max(e, axis=1, keepdims=True)
        p = jnp.exp(e - m)
    out = jnp.dot(p, h, preferred_element_type=jnp.float32)
    out = out / jnp.sum(p, axis=1, keepdims=True)
    if acc is not None:
        out = out + acc
    return jnp.maximum(out, 0.0)


_LBM = 256
_LSTEPS = [n // _LBM for n in _NS]
_BMS0 = [256, 512, 512, 256]
_STEPS0 = [_NS[j] // _BMS0[j] for j in range(4)]


def _a_body(lsts, sts, *refs):
    it = iter(refs)
    e_refs = [next(it) for _ in range(4)]
    wl_ref = next(it)
    bl_ref = next(it)
    wc0_ref = next(it)
    bc0_ref = next(it)
    a_ref = next(it)
    lap_refs = [next(it) for _ in range(4)]
    blow_refs = {j: next(it) for j in (1, 2, 3)}
    bup_refs = {j: next(it) for j in (0, 1, 2)}
    wn_ref = next(it)
    bn_ref = next(it)
    oh_refs = [next(it) for _ in range(4)]
    oy_refs = [next(it) for _ in range(4)]
    m_refs = [next(it) for _ in range(4)]
    hS = [next(it) for _ in range(4)]
    yS = [next(it) for _ in range(4)]

    t = pl.program_id(0)
    for j in range(4):
        @pl.when((t >= lsts[j]) & (t < lsts[j] + _LSTEPS[j]))
        def _(j=j):
            r = t - lsts[j]
            x = jnp.dot(e_refs[j][...], wl_ref[...], preferred_element_type=jnp.float32)
            x = x + bl_ref[...]
            oc = jnp.dot(x, wc0_ref[...], preferred_element_type=jnp.float32) + bc0_ref[...]
            hS[j][pl.ds(r * _LBM, _LBM), :] = oc[:, :_H]
            yS[j][pl.ds(r * _LBM, _LBM), :] = oc[:, _H:].astype(jnp.bfloat16)

    for j in range(4):
        @pl.when((t >= sts[j]) & (t < sts[j] + _STEPS0[j]))
        def _(j=j):
            bm = _BMS0[j]
            r = t - sts[j]
            h = hS[j][...]
            hb = hS[j][pl.ds(r * bm, bm), :]
            nz = lap_refs[j][...] != 0
            m_refs[j][...] = nz.astype(jnp.int8)
            x = _attn_math(
                a_ref[...], h, hb, nz,
                yS[j - 1][:, :_H] if j > 0 else None,
                yS[j + 1][:, _H:] if j < 3 else None,
                blow_refs[j][...] if j > 0 else None,
                bup_refs[j][...] if j < 3 else None,
                bound_softmax=False,
            )
            oc = jnp.dot(x, wn_ref[...], preferred_element_type=jnp.float32) + bn_ref[...]
            oh_refs[j][...] = oc[:, :_H]
            oy_refs[j][...] = oc[:, _H:].astype(jnp.bfloat16)


def _a_stage(embs, w_lin, b_lin2, wc0, bc0, a2, laps, bnds, wn, bn):
    c = embs[0].shape[1]
    lsts, lin_t = _starts(_LSTEPS)
    sts0, r0_t = _starts(_STEPS0)
    sts = [lin_t + s for s in sts0]
    total = lin_t + r0_t
    in_specs = (
        [pl.BlockSpec((_LBM, c), _win_row(lsts[j], _LSTEPS[j] - 1)) for j in range(4)]
        + [
            pl.BlockSpec((c, _H), _const2(0, 0)),
            pl.BlockSpec((1, _H), _const2(0, 0)),
            pl.BlockSpec((_H, _HC), _const2(0, 0)),
            pl.BlockSpec((1, _HC), _const2(0, 0)),
            pl.BlockSpec((2, _H), _const2(0, 0)),
        ]
        + [pl.BlockSpec((_BMS0[j], _NS[j]), _win_row(sts[j], _STEPS0[j] - 1)) for j in range(4)]
        + [pl.BlockSpec((_NS[j - 1], _BMS0[j]), _win_col(sts[j], _STEPS0[j] - 1)) for j in (1, 2, 3)]
        + [pl.BlockSpec((_BMS0[j], _NS[j + 1]), _win_row(sts[j], _STEPS0[j] - 1)) for j in (0, 1, 2)]
        + [
            pl.BlockSpec((_H, _HC), _const2(0, 0)),
            pl.BlockSpec((1, _HC), _const2(0, 0)),
        ]
    )
    args = (
        list(embs)
        + [w_lin, b_lin2, wc0, bc0, a2]
        + list(laps)
        + [bnds[j] for j in (1, 2, 3)]
        + [bnds[j + 1] for j in (0, 1, 2)]
        + [wn, bn]
    )
    out_specs = (
        [pl.BlockSpec((_BMS0[j], _H), _win_row(sts[j], _STEPS0[j] - 1)) for j in range(4)]
        + [pl.BlockSpec((_BMS0[j], 2 * _H), _win_row(sts[j], _STEPS0[j] - 1)) for j in range(4)]
        + [pl.BlockSpec((_BMS0[j], _NS[j]), _win_row(sts[j], _STEPS0[j] - 1)) for j in range(4)]
    )
    out_shape = (
        [jax.ShapeDtypeStruct((n, _H), jnp.float32) for n in _NS]
        + [jax.ShapeDtypeStruct((n, 2 * _H), jnp.bfloat16) for n in _NS]
        + [jax.ShapeDtypeStruct((n, n), jnp.int8) for n in _NS]
    )
    scratch = (
        [pltpu.VMEM((n, _H), jnp.float32) for n in _NS]
        + [pltpu.VMEM((n, 2 * _H), jnp.bfloat16) for n in _NS]
    )
    res = pl.pallas_call(
        functools.partial(_a_body, lsts, sts),
        grid=(total,),
        in_specs=list(in_specs),
        out_specs=list(out_specs),
        out_shape=list(out_shape),
        scratch_shapes=scratch,
    )(*args)
    return list(res[:4]), list(res[4:8]), list(res[8:12])


_BMS = [512, 512, 512, 256]
_RSTEPS = [_NS[j] // _BMS[j] for j in range(4)]


def _b_body(sts, rt, *refs):
    it = iter(refs)
    h_refs = [next(it) for _ in range(4)]
    a_ref = next(it)
    wc_ref = next(it)
    bc_ref = next(it)
    m_refs = [next(it) for _ in range(4)]
    blow_refs = {j: next(it) for j in (1, 2, 3)}
    ylow_refs = {j: next(it) for j in (1, 2, 3)}
    bup_refs = {j: next(it) for j in (0, 1, 2)}
    yup_refs = {j: next(it) for j in (0, 1, 2)}
    oh_refs = [next(it) for _ in range(4)]
    oy_refs = [next(it) for _ in range(4)]
    hS = [next(it) for _ in range(4)]
    yS = [next(it) for _ in range(4)]

    t = pl.program_id(0)
    for j in range(4):
        @pl.when((t >= sts[j]) & (t < sts[j] + _RSTEPS[j]))
        def _(j=j):
            bm = _BMS[j]
            r = t - sts[j]
            x = _attn_math(
                a_ref[0], h_refs[j][...], h_refs[j][pl.ds(r * bm, bm), :],
                m_refs[j][...] != 0,
                ylow_refs[j][...] if j > 0 else None,
                yup_refs[j][...] if j < 3 else None,
                blow_refs[j][...] if j > 0 else None,
                bup_refs[j][...] if j < 3 else None,
            )
            oc = jnp.dot(x, wc_ref[0], preferred_element_type=jnp.float32) + bc_ref[0]
            hS[j][pl.ds(r * bm, bm), :] = oc[:, :_H]
            yS[j][pl.ds(r * bm, bm), :] = oc[:, _H:].astype(jnp.bfloat16)

    for j in range(4):
        @pl.when((t >= rt + sts[j]) & (t < rt + sts[j] + _RSTEPS[j]))
        def _(j=j):
            bm = _BMS[j]
            r = t - rt - sts[j]
            x = _attn_math(
                a_ref[0], hS[j][...], hS[j][pl.ds(r * bm, bm), :],
                m_refs[j][...] != 0,
                yS[j - 1][:, :_H] if j > 0 else None,
                yS[j + 1][:, _H:] if j < 3 else None,
                blow_refs[j][...] if j > 0 else None,
                bup_refs[j][...] if j < 3 else None,
            )
            oc = jnp.dot(x, wc_ref[0], preferred_element_type=jnp.float32) + bc_ref[0]
            oh_refs[j][...] = oc[:, :_H]
            oy_refs[j][...] = oc[:, _H:].astype(jnp.bfloat16)


def _b_stage(hs, ys, a12, wc12, bc12, masks, bnds):
    sts, rt = _starts(_RSTEPS)
    total = 2 * rt
    in_specs = (
        [pl.BlockSpec((_NS[j], _H), _const2(0, 0)) for j in range(4)]
        + [
            pl.BlockSpec((1, 2, _H), _rsel3(rt, 1)),
            pl.BlockSpec((1, _H, _HC), _rsel3(rt, 1)),
            pl.BlockSpec((1, 1, _HC), _rsel3(rt, 1)),
        ]
        + [pl.BlockSpec((_BMS[j], _NS[j]), _pwin_row(sts[j], _RSTEPS[j] - 1, rt)) for j in range(4)]
        + [pl.BlockSpec((_NS[j - 1], _BMS[j]), _pwin_col(sts[j], _RSTEPS[j] - 1, rt)) for j in (1, 2, 3)]
        + [pl.BlockSpec((_NS[j - 1], _H), _const2(0, 0)) for j in (1, 2, 3)]
        + [pl.BlockSpec((_BMS[j], _NS[j + 1]), _pwin_row(sts[j], _RSTEPS[j] - 1, rt)) for j in (0, 1, 2)]
        + [pl.BlockSpec((_NS[j + 1], _H), _const2(0, 1)) for j in (0, 1, 2)]
    )
    args = (
        list(hs)
        + [a12, wc12, bc12]
        + list(masks)
        + [bnds[j] for j in (1, 2, 3)]
        + [ys[j - 1] for j in (1, 2, 3)]
        + [bnds[j + 1] for j in (0, 1, 2)]
        + [ys[j + 1] for j in (0, 1, 2)]
    )
    out_specs = (
        [pl.BlockSpec((_BMS[j], _H), _win_row(rt + sts[j], _RSTEPS[j] - 1)) for j in range(4)]
        + [pl.BlockSpec((_BMS[j], 2 * _H), _win_row(rt + sts[j], _RSTEPS[j] - 1)) for j in range(4)]
    )
    out_shape = (
        [jax.ShapeDtypeStruct((n, _H), jnp.float32) for n in _NS]
        + [jax.ShapeDtypeStruct((n, 2 * _H), jnp.bfloat16) for n in _NS]
    )
    scratch = (
        [pltpu.VMEM((n, _H), jnp.float32) for n in _NS]
        + [pltpu.VMEM((n, 2 * _H), jnp.bfloat16) for n in _NS]
    )
    res = pl.pallas_call(
        functools.partial(_b_body, sts, rt),
        grid=(total,),
        in_specs=list(in_specs),
        out_specs=list(out_specs),
        out_shape=list(out_shape),
        scratch_shapes=scratch,
    )(*args)
    return list(res[:4]), list(res[4:8])


def _c_body(sts, *refs):
    it = iter(refs)
    h_refs = [next(it) for _ in range(4)]
    a_ref = next(it)
    m_refs = [next(it) for _ in range(4)]
    blow_refs = {j: next(it) for j in (1, 2, 3)}
    ylow_refs = {j: next(it) for j in (1, 2, 3)}
    bup_refs = {j: next(it) for j in (0, 1, 2)}
    yup_refs = {j: next(it) for j in (0, 1, 2)}
    s_refs = [next(it) for _ in range(4)]
    wr_ref = next(it)
    br_ref = next(it)
    o_ref = next(it)
    acc_ref = next(it)

    t = pl.program_id(0)
    for j in range(4):
        @pl.when((t >= sts[j]) & (t < sts[j] + _RSTEPS[j]))
        def _(j=j):
            bm = _BMS[j]
            r = t - sts[j]
            x = _attn_math(
                a_ref[...], h_refs[j][...], h_refs[j][pl.ds(r * bm, bm), :],
                m_refs[j][...] != 0,
                ylow_refs[j][...] if j > 0 else None,
                yup_refs[j][...] if j < 3 else None,
                blow_refs[j][...] if j > 0 else None,
                bup_refs[j][...] if j < 3 else None,
            )
            s_blk = s_refs[j][:, pl.ds(r * bm, bm)]
            part = jnp.dot(s_blk, x, preferred_element_type=jnp.float32)

            @pl.when(r == 0)
            def _():
                acc_ref[2 * j:2 * j + 2, :] = part

            @pl.when(r > 0)
            def _():
                acc_ref[2 * j:2 * j + 2, :] = acc_ref[2 * j:2 * j + 2, :] + part

    @pl.when(t == sts[3] + _RSTEPS[3])
    def _():
        acc = acc_ref[...]
        ps = acc[0:2] + acc[2:4] + acc[4:6] + acc[6:8]
        feat = ps.reshape(1, 2 * _H)
        o_ref[...] = jnp.dot(feat, wr_ref[...], preferred_element_type=jnp.float32) + br_ref[...]


def _c_stage(hs, ys, a2, masks, bnds, ss, w_rel, b_rel):
    sts, rt = _starts(_RSTEPS)
    total = rt + 1
    in_specs = (
        [pl.BlockSpec((_NS[j], _H), _const2(0, 0)) for j in range(4)]
        + [pl.BlockSpec((2, _H), _const2(0, 0))]
        + [pl.BlockSpec((_BMS[j], _NS[j]), _win_row(sts[j], _RSTEPS[j] - 1)) for j in range(4)]
        + [pl.BlockSpec((_NS[j - 1], _BMS[j]), _win_col(sts[j], _RSTEPS[j] - 1)) for j in (1, 2, 3)]
        + [pl.BlockSpec((_NS[j - 1], _H), _const2(0, 0)) for j in (1, 2, 3)]
        + [pl.BlockSpec((_BMS[j], _NS[j + 1]), _win_row(sts[j], _RSTEPS[j] - 1)) for j in (0, 1, 2)]
        + [pl.BlockSpec((_NS[j + 1], _H), _const2(0, 1)) for j in (0, 1, 2)]
        + [pl.BlockSpec((2, _NS[j]), _const2(0, 0)) for j in range(4)]
        + [
            pl.BlockSpec(w_rel.shape, _const2(0, 0)),
            pl.BlockSpec((1, b_rel.shape[-1]), _const2(0, 0)),
        ]
    )
    args = (
        list(hs)
        + [a2]
        + list(masks)
        + [bnds[j] for j in (1, 2, 3)]
        + [ys[j - 1] for j in (1, 2, 3)]
        + [bnds[j + 1] for j in (0, 1, 2)]
        + [ys[j + 1] for j in (0, 1, 2)]
        + list(ss)
        + [w_rel, b_rel]
    )
    out = pl.pallas_call(
        functools.partial(_c_body, sts),
        grid=(total,),
        in_specs=list(in_specs),
        out_specs=pl.BlockSpec((1, b_rel.shape[-1]), _const2(0, 0)),
        out_shape=jax.ShapeDtypeStruct((1, b_rel.shape[-1]), jnp.float32),
        scratch_shapes=[pltpu.VMEM((8, _H), jnp.float32)],
    )(*args)
    return out


def kernel(emb0, emb1, emb2, emb3, lap0, lap1, lap2, lap3, bnd1, bnd2, bnd3, params, order, idx, rel):
    embs = [emb0, emb1, emb2, emb3]
    laps = [lap0, lap1, lap2, lap3]
    bnds = [None] + [b.astype(jnp.bfloat16) for b in (bnd1, bnd2, bnd3)]
    lay = params["layers"]
    wcats = [jnp.concatenate([l["W"], l["W_low"], l["W_up"]], axis=1) for l in lay]
    bcats = [
        jnp.concatenate([l["b"], jnp.zeros((2 * _H,), jnp.float32)]).reshape(1, _HC)
        for l in lay
    ]
    a2s = [jnp.concatenate([l["a_src"].T, l["a_dst"].T], axis=0) for l in lay]
    b_lin2 = params["b_lin"].reshape(1, _H)

    hs, ys, masks = _a_stage(
        embs, params["W_lin"], b_lin2, wcats[0], bcats[0], a2s[0], laps, bnds,
        wcats[1], bcats[1],
    )

    a12 = jnp.stack([a2s[1], a2s[2]])
    wc12 = jnp.stack([wcats[2], wcats[3]])
    bc12 = jnp.stack([bcats[2], bcats[3]])
    hs, ys = _b_stage(hs, ys, a12, wc12, bc12, masks, bnds)

    ss = []
    for j in range(4):
        n = _NS[j]
        sel = jnp.where(order == j, 1.0, 0.0)
        onehot = jnp.where(jnp.arange(n, dtype=jnp.int32) == idx, sel, 0.0)
        ss.append(jnp.stack([jnp.ones((n,), jnp.float32), onehot]))
    out = _c_stage(hs, ys, a2s[3], masks, bnds, ss,
                   params["W_rel"], params["b_rel"].reshape(1, -1))
    nz = jnp.nonzero(rel, size=out.shape[1])[0]
    return out[0][nz]


# speedup vs baseline: 1.0206x; 1.0206x over previous
import functools

import jax
import jax.numpy as jnp
from jax.experimental import pallas as pl
from jax.experimental.pallas import tpu as pltpu

_NS = [1024, 2048, 1536, 512]
_H = 256
_HC = 3 * _H


def _starts(steps):
    s, acc = [], 0
    for v in steps:
        s.append(acc)
        acc += v
    return s, acc


def _win_row(start, last):
    return lambda t: (jnp.clip(t - start, 0, last), 0)


def _win_col(start, last):
    return lambda t: (0, jnp.clip(t - start, 0, last))


def _pwin_row(start, last, period):
    return lambda t: (jnp.clip(t % period - start, 0, last), 0)


def _pwin_col(start, last, period):
    return lambda t: (0, jnp.clip(t % period - start, 0, last))


def _const2(i, k):
    return lambda t, _i=i, _k=k: (_i, _k)


def _rsel3(period, hi):
    return lambda t: (jnp.clip(t // period, 0, hi), 0, 0)


def _attn_math(a, h, hb, nz, ylow, yup, blow, bup, bound_softmax=False):
    acc = None
    if blow is not None:
        acc = jax.lax.dot_general(
            blow, ylow,
            dimension_numbers=(((0,), (0,)), ((), ())),
            preferred_element_type=jnp.float32,
        )
    if bup is not None:
        up = jnp.dot(bup, yup, preferred_element_type=jnp.float32)
        acc = up if acc is None else acc + up
    s_dst = jnp.sum(h * a[1:2, :], axis=1)[None, :]
    s_src = jnp.sum(hb * a[0:1, :], axis=1, keepdims=True)
    e = s_src + s_dst
    e = jnp.maximum(e, 0.2 * e)
    if bound_softmax:
        b = s_src + jnp.max(s_dst)
        b = jnp.maximum(b, 0.2 * b)
        p = jnp.where(nz, jnp.exp(e - b), 0.0)
    else:
        e = jnp.where(nz, e, -1e9)
        m = jnp.max(e, axis=1, keepdims=True)
        p = jnp.exp(e - m)
    out = jnp.dot(p, h, preferred_element_type=jnp.float32)
    out = out / jnp.sum(p, axis=1, keepdims=True)
    if acc is not None:
        out = out + acc
    return jnp.maximum(out, 0.0)


_LBM = 256
_LSTEPS = [n // _LBM for n in _NS]
_BMS0 = [256, 512, 512, 256]
_STEPS0 = [_NS[j] // _BMS0[j] for j in range(4)]


def _a_body(lsts, sts, *refs):
    it = iter(refs)
    e_refs = [next(it) for _ in range(4)]
    wl_ref = next(it)
    bl_ref = next(it)
    wc0_ref = next(it)
    bc0_ref = next(it)
    a_ref = next(it)
    lap_refs = [next(it) for _ in range(4)]
    blow_refs = {j: next(it) for j in (1, 2, 3)}
    bup_refs = {j: next(it) for j in (0, 1, 2)}
    wn_ref = next(it)
    bn_ref = next(it)
    oh_refs = [next(it) for _ in range(4)]
    oy_refs = [next(it) for _ in range(4)]
    m_refs = [next(it) for _ in range(4)]
    hS = [next(it) for _ in range(4)]
    yS = [next(it) for _ in range(4)]

    t = pl.program_id(0)
    for j in range(4):
        @pl.when((t >= lsts[j]) & (t < lsts[j] + _LSTEPS[j]))
        def _(j=j):
            r = t - lsts[j]
            x = jnp.dot(e_refs[j][...], wl_ref[...], preferred_element_type=jnp.float32)
            x = x + bl_ref[...]
            oc = jnp.dot(x, wc0_ref[...], preferred_element_type=jnp.float32) + bc0_ref[...]
            hS[j][pl.ds(r * _LBM, _LBM), :] = oc[:, :_H]
            yS[j][pl.ds(r * _LBM, _LBM), :] = oc[:, _H:].astype(jnp.bfloat16)

    for j in range(4):
        @pl.when((t >= sts[j]) & (t < sts[j] + _STEPS0[j]))
        def _(j=j):
            bm = _BMS0[j]
            r = t - sts[j]
            h = hS[j][...]
            hb = hS[j][pl.ds(r * bm, bm), :]
            nz = lap_refs[j][...] != 0
            m_refs[j][...] = nz.astype(jnp.int8)
            x = _attn_math(
                a_ref[...], h, hb, nz,
                yS[j - 1][:, :_H] if j > 0 else None,
                yS[j + 1][:, _H:] if j < 3 else None,
                blow_refs[j][...] if j > 0 else None,
                bup_refs[j][...] if j < 3 else None,
                bound_softmax=False,
            )
            oc = jnp.dot(x, wn_ref[...], preferred_element_type=jnp.float32) + bn_ref[...]
            oh_refs[j][...] = oc[:, :_H]
            oy_refs[j][...] = oc[:, _H:].astype(jnp.bfloat16)


def _a_stage(embs, w_lin, b_lin2, wc0, bc0, a2, laps, bnds, wn, bn):
    c = embs[0].shape[1]
    lsts, lin_t = _starts(_LSTEPS)
    sts0, r0_t = _starts(_STEPS0)
    sts = [lin_t + s for s in sts0]
    total = lin_t + r0_t
    in_specs = (
        [pl.BlockSpec((_LBM, c), _win_row(lsts[j], _LSTEPS[j] - 1)) for j in range(4)]
        + [
            pl.BlockSpec((c, _H), _const2(0, 0)),
            pl.BlockSpec((1, _H), _const2(0, 0)),
            pl.BlockSpec((_H, _HC), _const2(0, 0)),
            pl.BlockSpec((1, _HC), _const2(0, 0)),
            pl.BlockSpec((2, _H), _const2(0, 0)),
        ]
        + [pl.BlockSpec((_BMS0[j], _NS[j]), _win_row(sts[j], _STEPS0[j] - 1)) for j in range(4)]
        + [pl.BlockSpec((_NS[j - 1], _BMS0[j]), _win_col(sts[j], _STEPS0[j] - 1)) for j in (1, 2, 3)]
        + [pl.BlockSpec((_BMS0[j], _NS[j + 1]), _win_row(sts[j], _STEPS0[j] - 1)) for j in (0, 1, 2)]
        + [
            pl.BlockSpec((_H, _HC), _const2(0, 0)),
            pl.BlockSpec((1, _HC), _const2(0, 0)),
        ]
    )
    args = (
        list(embs)
        + [w_lin, b_lin2, wc0, bc0, a2]
        + list(laps)
        + [bnds[j] for j in (1, 2, 3)]
        + [bnds[j + 1] for j in (0, 1, 2)]
        + [wn, bn]
    )
    out_specs = (
        [pl.BlockSpec((_BMS0[j], _H), _win_row(sts[j], _STEPS0[j] - 1)) for j in range(4)]
        + [pl.BlockSpec((_BMS0[j], 2 * _H), _win_row(sts[j], _STEPS0[j] - 1)) for j in range(4)]
        + [pl.BlockSpec((_BMS0[j], _NS[j]), _win_row(sts[j], _STEPS0[j] - 1)) for j in range(4)]
    )
    out_shape = (
        [jax.ShapeDtypeStruct((n, _H), jnp.float32) for n in _NS]
        + [jax.ShapeDtypeStruct((n, 2 * _H), jnp.bfloat16) for n in _NS]
        + [jax.ShapeDtypeStruct((n, n), jnp.int8) for n in _NS]
    )
    scratch = (
        [pltpu.VMEM((n, _H), jnp.float32) for n in _NS]
        + [pltpu.VMEM((n, 2 * _H), jnp.bfloat16) for n in _NS]
    )
    res = pl.pallas_call(
        functools.partial(_a_body, lsts, sts),
        grid=(total,),
        in_specs=list(in_specs),
        out_specs=list(out_specs),
        out_shape=list(out_shape),
        scratch_shapes=scratch,
    )(*args)
    return list(res[:4]), list(res[4:8]), list(res[8:12])


_BMS = [512, 512, 512, 256]
_RSTEPS = [_NS[j] // _BMS[j] for j in range(4)]


def _b_body(sts, rt, *refs):
    it = iter(refs)
    h_refs = [next(it) for _ in range(4)]
    a_ref = next(it)
    wc_ref = next(it)
    bc_ref = next(it)
    m_refs = [next(it) for _ in range(4)]
    blow_refs = {j: next(it) for j in (1, 2, 3)}
    ylow_refs = {j: next(it) for j in (1, 2, 3)}
    bup_refs = {j: next(it) for j in (0, 1, 2)}
    yup_refs = {j: next(it) for j in (0, 1, 2)}
    oh_refs = [next(it) for _ in range(4)]
    oy_refs = [next(it) for _ in range(4)]
    hS = [next(it) for _ in range(4)]
    yS = [next(it) for _ in range(4)]

    t = pl.program_id(0)
    for j in range(4):
        @pl.when((t >= sts[j]) & (t < sts[j] + _RSTEPS[j]))
        def _(j=j):
            bm = _BMS[j]
            r = t - sts[j]
            x = _attn_math(
                a_ref[0], h_refs[j][...], h_refs[j][pl.ds(r * bm, bm), :],
                m_refs[j][...] != 0,
                ylow_refs[j][...] if j > 0 else None,
                yup_refs[j][...] if j < 3 else None,
                blow_refs[j][...] if j > 0 else None,
                bup_refs[j][...] if j < 3 else None,
            )
            oc = jnp.dot(x, wc_ref[0], preferred_element_type=jnp.float32) + bc_ref[0]
            hS[j][pl.ds(r * bm, bm), :] = oc[:, :_H]
            yS[j][pl.ds(r * bm, bm), :] = oc[:, _H:].astype(jnp.bfloat16)

    for j in range(4):
        @pl.when((t >= rt + sts[j]) & (t < rt + sts[j] + _RSTEPS[j]))
        def _(j=j):
            bm = _BMS[j]
            r = t - rt - sts[j]
            x = _attn_math(
                a_ref[0], hS[j][...], hS[j][pl.ds(r * bm, bm), :],
                m_refs[j][...] != 0,
                yS[j - 1][:, :_H] if j > 0 else None,
                yS[j + 1][:, _H:] if j < 3 else None,
                blow_refs[j][...] if j > 0 else None,
                bup_refs[j][...] if j < 3 else None,
            )
            oc = jnp.dot(x, wc_ref[0], preferred_element_type=jnp.float32) + bc_ref[0]
            oh_refs[j][...] = oc[:, :_H]
            oy_refs[j][...] = oc[:, _H:].astype(jnp.bfloat16)


def _b_stage(hs, ys, a12, wc12, bc12, masks, bnds):
    sts, rt = _starts(_RSTEPS)
    total = 2 * rt
    in_specs = (
        [pl.BlockSpec((_NS[j], _H), _const2(0, 0)) for j in range(4)]
        + [
            pl.BlockSpec((1, 2, _H), _rsel3(rt, 1)),
            pl.BlockSpec((1, _H, _HC), _rsel3(rt, 1)),
            pl.BlockSpec((1, 1, _HC), _rsel3(rt, 1)),
        ]
        + [pl.BlockSpec((_BMS[j], _NS[j]), _pwin_row(sts[j], _RSTEPS[j] - 1, rt)) for j in range(4)]
        + [pl.BlockSpec((_NS[j - 1], _BMS[j]), _pwin_col(sts[j], _RSTEPS[j] - 1, rt)) for j in (1, 2, 3)]
        + [pl.BlockSpec((_NS[j - 1], _H), _const2(0, 0)) for j in (1, 2, 3)]
        + [pl.BlockSpec((_BMS[j], _NS[j + 1]), _pwin_row(sts[j], _RSTEPS[j] - 1, rt)) for j in (0, 1, 2)]
        + [pl.BlockSpec((_NS[j + 1], _H), _const2(0, 1)) for j in (0, 1, 2)]
    )
    args = (
        list(hs)
        + [a12, wc12, bc12]
        + list(masks)
        + [bnds[j] for j in (1, 2, 3)]
        + [ys[j - 1] for j in (1, 2, 3)]
        + [bnds[j + 1] for j in (0, 1, 2)]
        + [ys[j + 1] for j in (0, 1, 2)]
    )
    out_specs = (
        [pl.BlockSpec((_BMS[j], _H), _win_row(rt + sts[j], _RSTEPS[j] - 1)) for j in range(4)]
        + [pl.BlockSpec((_BMS[j], 2 * _H), _win_row(rt + sts[j], _RSTEPS[j] - 1)) for j in range(4)]
    )
    out_shape = (
        [jax.ShapeDtypeStruct((n, _H), jnp.float32) for n in _NS]
        + [jax.ShapeDtypeStruct((n, 2 * _H), jnp.bfloat16) for n in _NS]
    )
    scratch = (
        [pltpu.VMEM((n, _H), jnp.float32) for n in _NS]
        + [pltpu.VMEM((n, 2 * _H), jnp.bfloat16) for n in _NS]
    )
    res = pl.pallas_call(
        functools.partial(_b_body, sts, rt),
        grid=(total,),
        in_specs=list(in_specs),
        out_specs=list(out_specs),
        out_shape=list(out_shape),
        scratch_shapes=scratch,
    )(*args)
    return list(res[:4]), list(res[4:8])


def _c_body(sts, *refs):
    it = iter(refs)
    h_refs = [next(it) for _ in range(4)]
    a_ref = next(it)
    m_refs = [next(it) for _ in range(4)]
    blow_refs = {j: next(it) for j in (1, 2, 3)}
    ylow_refs = {j: next(it) for j in (1, 2, 3)}
    bup_refs = {j: next(it) for j in (0, 1, 2)}
    yup_refs = {j: next(it) for j in (0, 1, 2)}
    s_refs = [next(it) for _ in range(4)]
    wr_ref = next(it)
    br_ref = next(it)
    o_ref = next(it)
    acc_ref = next(it)

    t = pl.program_id(0)
    for j in range(4):
        @pl.when((t >= sts[j]) & (t < sts[j] + _RSTEPS[j]))
        def _(j=j):
            bm = _BMS[j]
            r = t - sts[j]
            x = _attn_math(
                a_ref[...], h_refs[j][...], h_refs[j][pl.ds(r * bm, bm), :],
                m_refs[j][...] != 0,
                ylow_refs[j][...] if j > 0 else None,
                yup_refs[j][...] if j < 3 else None,
                blow_refs[j][...] if j > 0 else None,
                bup_refs[j][...] if j < 3 else None,
            )
            s_blk = s_refs[j][:, pl.ds(r * bm, bm)]
            part = jnp.dot(s_blk, x, preferred_element_type=jnp.float32)

            @pl.when(r == 0)
            def _():
                acc_ref[2 * j:2 * j + 2, :] = part

            @pl.when(r > 0)
            def _():
                acc_ref[2 * j:2 * j + 2, :] = acc_ref[2 * j:2 * j + 2, :] + part

    @pl.when(t == sts[3] + _RSTEPS[3])
    def _():
        acc = acc_ref[...]
        ps = acc[0:2] + acc[2:4] + acc[4:6] + acc[6:8]
        feat = ps.reshape(1, 2 * _H)
        o_ref[...] = jnp.dot(feat, wr_ref[...], preferred_element_type=jnp.float32) + br_ref[...]


def _c_stage(hs, ys, a2, masks, bnds, ss, w_rel, b_rel):
    sts, rt = _starts(_RSTEPS)
    total = rt + 1
    in_specs = (
        [pl.BlockSpec((_NS[j], _H), _const2(0, 0)) for j in range(4)]
        + [pl.BlockSpec((2, _H), _const2(0, 0))]
        + [pl.BlockSpec((_BMS[j], _NS[j]), _win_row(sts[j], _RSTEPS[j] - 1)) for j in range(4)]
        + [pl.BlockSpec((_NS[j - 1], _BMS[j]), _win_col(sts[j], _RSTEPS[j] - 1)) for j in (1, 2, 3)]
        + [pl.BlockSpec((_NS[j - 1], _H), _const2(0, 0)) for j in (1, 2, 3)]
        + [pl.BlockSpec((_BMS[j], _NS[j + 1]), _win_row(sts[j], _RSTEPS[j] - 1)) for j in (0, 1, 2)]
        + [pl.BlockSpec((_NS[j + 1], _H), _const2(0, 1)) for j in (0, 1, 2)]
        + [pl.BlockSpec((2, _NS[j]), _const2(0, 0)) for j in range(4)]
        + [
            pl.BlockSpec(w_rel.shape, _const2(0, 0)),
            pl.BlockSpec((1, b_rel.shape[-1]), _const2(0, 0)),
        ]
    )
    args = (
        list(hs)
        + [a2]
        + list(masks)
        + [bnds[j] for j in (1, 2, 3)]
        + [ys[j - 1] for j in (1, 2, 3)]
        + [bnds[j + 1] for j in (0, 1, 2)]
        + [ys[j + 1] for j in (0, 1, 2)]
        + list(ss)
        + [w_rel, b_rel]
    )
    out = pl.pallas_call(
        functools.partial(_c_body, sts),
        grid=(total,),
        in_specs=list(in_specs),
        out_specs=pl.BlockSpec((1, b_rel.shape[-1]), _const2(0, 0)),
        out_shape=jax.ShapeDtypeStruct((1, b_rel.shape[-1]), jnp.float32),
        scratch_shapes=[pltpu.VMEM((8, _H), jnp.float32)],
    )(*args)
    return out


def kernel(emb0, emb1, emb2, emb3, lap0, lap1, lap2, lap3, bnd1, bnd2, bnd3, params, order, idx, rel):
    embs = [emb0, emb1, emb2, emb3]
    laps = [lap0, lap1, lap2, lap3]
    bnds = [None] + [b.astype(jnp.bfloat16) for b in (bnd1, bnd2, bnd3)]
    lay = params["layers"]
    wcats = [jnp.concatenate([l["W"], l["W_low"], l["W_up"]], axis=1) for l in lay]
    bcats = [
        jnp.concatenate([l["b"], jnp.zeros((2 * _H,), jnp.float32)]).reshape(1, _HC)
        for l in lay
    ]
    a2s = [jnp.concatenate([l["a_src"].T, l["a_dst"].T], axis=0) for l in lay]
    b_lin2 = params["b_lin"].reshape(1, _H)

    hs, ys, masks = _a_stage(
        embs, params["W_lin"], b_lin2, wcats[0], bcats[0], a2s[0], laps, bnds,
        wcats[1], bcats[1],
    )

    a12 = jnp.stack([a2s[1], a2s[2]])
    wc12 = jnp.stack([wcats[2], wcats[3]])
    bc12 = jnp.stack([bcats[2], bcats[3]])
    hs, ys = _b_stage(hs, ys, a12, wc12, bc12, masks, bnds)

    ss = []
    for j in range(4):
        n = _NS[j]
        sel = jnp.where(order == j, 1.0, 0.0)
        onehot = jnp.where(jnp.arange(n, dtype=jnp.int32) == idx, sel, 0.0)
        ss.append(jnp.stack([jnp.ones((n,), jnp.float32), onehot]))
    out = _c_stage(hs, ys, a2s[3], masks, bnds, ss,
                   params["W_rel"], params["b_rel"].reshape(1, -1))
    nz = jnp.nonzero(rel, size=out.shape[1])[0]
    return out[0][nz]
